# baseline (speedup 1.0000x reference)
import jax
import jax.numpy as jnp
import numpy as np
from jax.experimental import pallas as pl
from jax.experimental.pallas import tpu as pltpu

S, B, D = 8192, 64, 64
N_SAMPLE = int(S * 0.1)
GB = 8
NB = B // GB
_MININT = -2147483648

_G_CACHE = [None]


def _gumbel_bs():
    if _G_CACHE[0] is None:
        try:
            with jax.ensure_compile_time_eval():
                g = jax.random.gumbel(jax.random.key(42), (B, S),
                                      dtype=jnp.float32)
            _G_CACHE[0] = np.asarray(g)
        except Exception:
            return jax.random.gumbel(jax.random.key(42), (B, S),
                                     dtype=jnp.float32)
    return _G_CACHE[0]


def _order_i32(x):
    m = jax.lax.bitcast_convert_type(x, jnp.int32)
    return jnp.where(m < 0, m ^ jnp.int32(0x7FFFFFFF), m)


def _task_attention_kernel(qt_ref, kv_ref, g_ref, out_ref, keys_ref):
    i = pl.program_id(0)
    w = jax.lax.dot_general(
        kv_ref[...], qt_ref[...].astype(jnp.bfloat16),
        (((1,), (0,)), ((), ())),
        preferred_element_type=jnp.float32,
    )
    keys_ref[pl.ds(i * GB, GB), :] = _order_i32(g_ref[...] - w)

    @pl.when(i == NB - 1)
    def _select_and_mask():
        okeys = keys_ref[...]

        tx = jnp.zeros((B, 1), jnp.int32)
        for j in range(32):
            cand_x = tx | jnp.int32(1 << (31 - j) if j else _MININT)
            cand_s = cand_x ^ jnp.int32(_MININT)
            cnt = jnp.sum((okeys >= cand_s).astype(jnp.int32), axis=1,
                          keepdims=True)
            tx = jnp.where(cnt >= N_SAMPLE, cand_x, tx)
        thresh = tx ^ jnp.int32(_MININT)
        out_ref[...] = jnp.where(okeys >= thresh, 0.0, 1.0)


@jax.jit
def kernel(q, k, lengths):
    del lengths
    qt = jnp.transpose(q, (1, 2, 0)).reshape(B * D, S)
    kv = k[:, :, 0]
    sel = jnp.eye(GB, dtype=jnp.float32)[jnp.arange(B) % GB]
    k3 = (sel[:, :, None] * kv[:, None, :]).reshape(B, GB * D)
    k3 = k3.astype(jnp.bfloat16)
    g_bs = _gumbel_bs()

    mask = pl.pallas_call(
        _task_attention_kernel,
        grid=(NB,),
        in_specs=[
            pl.BlockSpec((GB * D, S), lambda i: (i, 0)),
            pl.BlockSpec((GB, GB * D), lambda i: (i, 0)),
            pl.BlockSpec((GB, S), lambda i: (i, 0)),
        ],
        out_specs=pl.BlockSpec((B, S), lambda i: (0, 0)),
        out_shape=jax.ShapeDtypeStruct((B, S), jnp.float32),
        scratch_shapes=[pltpu.VMEM((B, S), jnp.int32)],
    )(qt, k3, g_bs)
    return jnp.transpose(mask)[:, :, None]


# speedup vs baseline: 1.0343x; 1.0343x over previous
import jax
import jax.numpy as jnp
import numpy as np
from jax.experimental import pallas as pl
from jax.experimental.pallas import tpu as pltpu

S, B, D = 8192, 64, 64
N_SAMPLE = int(S * 0.1)
GB = 8
NB = B // GB
_MININT = -2147483648

_G_CACHE = [None]


def _gumbel_bs():
    if _G_CACHE[0] is None:
        try:
            with jax.ensure_compile_time_eval():
                g = jax.random.gumbel(jax.random.key(42), (B, S),
                                      dtype=jnp.float32)
            _G_CACHE[0] = np.asarray(g)
        except Exception:
            return jax.random.gumbel(jax.random.key(42), (B, S),
                                     dtype=jnp.float32)
    return _G_CACHE[0]


def _order_i32(x):
    m = jax.lax.bitcast_convert_type(x, jnp.int32)
    return jnp.where(m < 0, m ^ jnp.int32(0x7FFFFFFF), m)


def _task_attention_kernel(qt_ref, kv_ref, g_ref, out_ref, keys_ref):
    i = pl.program_id(0)
    kvb = kv_ref[...].astype(jnp.bfloat16)
    col_b = jax.lax.broadcasted_iota(jnp.int32, (GB, GB * D), 1) // D
    row_b = jax.lax.broadcasted_iota(jnp.int32, (GB, GB * D), 0)
    k3 = jnp.where(col_b == row_b,
                   jnp.tile(kvb, (1, GB)), jnp.bfloat16(0.0))
    w = jax.lax.dot_general(
        k3, qt_ref[...].astype(jnp.bfloat16),
        (((1,), (0,)), ((), ())),
        preferred_element_type=jnp.float32,
    )
    keys_ref[pl.ds(i * GB, GB), :] = _order_i32(g_ref[...] - w)

    @pl.when(i == NB - 1)
    def _select_and_mask():
        okeys = keys_ref[...]

        tx = jnp.zeros((B, 1), jnp.int32)
        for j in range(32):
            cand_x = tx | jnp.int32(1 << (31 - j) if j else _MININT)
            cand_s = cand_x ^ jnp.int32(_MININT)
            cnt = jnp.sum((okeys >= cand_s).astype(jnp.int32), axis=1,
                          keepdims=True)
            tx = jnp.where(cnt >= N_SAMPLE, cand_x, tx)
        thresh = tx ^ jnp.int32(_MININT)
        out_ref[...] = jnp.where(okeys >= thresh, 0.0, 1.0)


@jax.jit
def kernel(q, k, lengths):
    del lengths
    qt = jnp.transpose(q, (1, 2, 0)).reshape(B * D, S)
    kv = k[:, :, 0]
    g_bs = _gumbel_bs()

    mask = pl.pallas_call(
        _task_attention_kernel,
        grid=(NB,),
        in_specs=[
            pl.BlockSpec((GB * D, S), lambda i: (i, 0)),
            pl.BlockSpec((GB, D), lambda i: (i, 0)),
            pl.BlockSpec((GB, S), lambda i: (i, 0)),
        ],
        out_specs=pl.BlockSpec((B, S), lambda i: (0, 0)),
        out_shape=jax.ShapeDtypeStruct((B, S), jnp.float32),
        scratch_shapes=[pltpu.VMEM((B, S), jnp.int32)],
    )(qt, kv, g_bs)
    return jnp.transpose(mask)[:, :, None]
